# baseline (speedup 1.0000x reference)
import functools

import jax
import jax.numpy as jnp
from jax import lax
from jax.experimental import pallas as pl
from jax.experimental.pallas import tpu as pltpu
from jax.experimental.pallas import tpu_sc as plsc

B = 16384
D = 64
P = 512
K = 8

TC_BLK = 2048

_NC = 2
_NS = 16
_NW = _NC * _NS
_RPW = B // _NW
_CH = 64
_NCH = _RPW // _CH
_CT = _CH // 8


def _sim_body(q_ref, p_ref, sim_ref):
    q = q_ref[...]
    pr = p_ref[...]
    qn = q * lax.rsqrt(jnp.maximum(jnp.sum(q * q, axis=1, keepdims=True),
                                   jnp.float32(1e-24)))
    pn = pr * lax.rsqrt(jnp.maximum(jnp.sum(pr * pr, axis=1, keepdims=True),
                                    jnp.float32(1e-24)))
    sim = lax.dot_general(
        qn, pn, (((1,), (1,)), ((), ())),
        preferred_element_type=jnp.float32)
    for j in range(P // 128):
        sim_ref[:, j] = sim[:, 128 * j:128 * (j + 1)].reshape(
            TC_BLK // 8, 8, 128)


def _similarity(query, prototypes):
    return pl.pallas_call(
        _sim_body,
        grid=(B // TC_BLK,),
        in_specs=[
            pl.BlockSpec((TC_BLK, D), lambda i: (i, 0)),
            pl.BlockSpec((P, D), lambda i: (0, 0)),
        ],
        out_specs=pl.BlockSpec((TC_BLK // 8, P // 128, 8, 128),
                               lambda i: (i, 0, 0, 0)),
        out_shape=jax.ShapeDtypeStruct((B // 8, P // 128, 8, 128),
                                       jnp.float32),
    )(query, prototypes)


@functools.partial(
    pl.kernel,
    out_type=jax.ShapeDtypeStruct((B, D), jnp.float32),
    mesh=plsc.VectorSubcoreMesh(core_axis_name="c", subcore_axis_name="s"),
    compiler_params=pltpu.CompilerParams(
        needs_layout_passes=False, use_tc_tiling_on_sc=False),
    scratch_types=[
        pltpu.VMEM((P, D), jnp.float32),
        pltpu.VMEM((_CT, P // 128, 8, 128), jnp.float32),
        pltpu.VMEM((_CT, P // 128, 8, 128), jnp.float32),
        pltpu.VMEM((_CH, D), jnp.float32),
        pltpu.VMEM((_CH, D), jnp.float32),
        pltpu.SemaphoreType.DMA,
        pltpu.SemaphoreType.DMA,
        pltpu.SemaphoreType.DMA,
        pltpu.SemaphoreType.DMA,
    ],
)
def _sc_topk(sim_hbm, protos_hbm, out_hbm, protos_v,
             simv0, simv1, outv0, outv1, si0, si1, so0, so1):
    wid = lax.axis_index("s") * _NC + lax.axis_index("c")
    pltpu.sync_copy(protos_hbm, protos_v)
    lane = lax.broadcasted_iota(jnp.int32, (16,), 0)
    lo8 = lane < 8

    simv = (simv0, simv1)
    outv = (outv0, outv1)
    sin = (si0, si1)
    son = (so0, so1)
    rt_base = wid * (_RPW // 8)

    def make_row_body(sbuf, obuf):
        def row_body(r):
            rt = jnp.right_shift(r, 3)
            s = jnp.bitwise_and(r, 7)
            NL = P // 16
            stack = []

            def node_desc(level, idx):
                if level == 5:
                    return True
                return idx % 2 == 0

            for c in range(NL):
                vals = sbuf[rt, c // 8, s, pl.ds((c % 8) * 16, 16)]
                sv, si = plsc.sort_key_val(vals, lane + 16 * c,
                                           descending=node_desc(0, c))
                cur = (0, c, sv, si)
                while stack and stack[-1][0] == cur[0]:
                    lvl, aidx, av, ai = stack.pop()
                    _, _, bv, bi = cur
                    mv = jnp.where(lo8, av, bv)
                    mi = jnp.where(lo8, ai, bi)
                    nl, nidx = lvl + 1, aidx // 2
                    sv, si = plsc.sort_key_val(
                        mv, mi, descending=node_desc(nl, nidx))
                    cur = (nl, nidx, sv, si)
                stack.append(cur)
            v, ix = stack[0][2], stack[0][3]

            e = jnp.where(lo8, jnp.exp(v - v[0]), jnp.float32(0.0))
            w = e / jnp.sum(e)

            accs = [jnp.zeros((16,), jnp.float32) for _ in range(D // 16)]
            for j in range(K):
                pj = jnp.broadcast_to(ix[j], (16,))
                wj = w[j]
                for c in range(D // 16):
                    g = plsc.load_gather(protos_v, [pj, lane + 16 * c])
                    accs[c] = accs[c] + wj * g
            for c in range(D // 16):
                obuf[r, pl.ds(16 * c, 16)] = accs[c]
        return row_body

    copies = [pltpu.async_copy(sim_hbm.at[pl.ds(rt_base, _CT)], simv0, si0),
              None]
    out_copies = [None, None]
    for ci in range(_NCH):
        cur = ci % 2
        nxt = 1 - cur
        if ci + 1 < _NCH:
            copies[nxt] = pltpu.async_copy(
                sim_hbm.at[pl.ds(rt_base + (ci + 1) * _CT, _CT)],
                simv[nxt], sin[nxt])
        copies[cur].wait()
        if out_copies[cur] is not None:
            out_copies[cur].wait()
        plsc.parallel_loop(0, _CH, unroll=1)(
            make_row_body(simv[cur], outv[cur]))
        out_copies[cur] = pltpu.async_copy(
            outv[cur], out_hbm.at[pl.ds(wid * _RPW + ci * _CH, _CH)],
            son[cur])
    for oc in out_copies:
        if oc is not None:
            oc.wait()


@jax.jit
def _run(query, prototypes):
    sim = _similarity(query, prototypes)
    return _sc_topk(sim, prototypes)


def kernel(query, prototypes, k):
    return _run(query, prototypes)


# speedup vs baseline: 1.0598x; 1.0598x over previous
import functools

import jax
import jax.numpy as jnp
from jax import lax
from jax.experimental import pallas as pl
from jax.experimental.pallas import tpu as pltpu
from jax.experimental.pallas import tpu_sc as plsc

B = 16384
D = 64
P = 512
K = 8

TC_BLK = 2048

_NC = 2
_NS = 16
_NW = _NC * _NS
_RPW = B // _NW
_CH = 64
_NCH = _RPW // _CH
_CT = _CH // 8


def _sim_body(q_ref, p_ref, sim_ref):
    q = q_ref[...]
    pr = p_ref[...]
    qn = q * lax.rsqrt(jnp.maximum(jnp.sum(q * q, axis=1, keepdims=True),
                                   jnp.float32(1e-24)))
    pn = pr * lax.rsqrt(jnp.maximum(jnp.sum(pr * pr, axis=1, keepdims=True),
                                    jnp.float32(1e-24)))
    sim = lax.dot_general(
        qn, pn, (((1,), (1,)), ((), ())),
        preferred_element_type=jnp.float32)
    for j in range(P // 128):
        sim_ref[:, j] = sim[:, 128 * j:128 * (j + 1)].reshape(
            TC_BLK // 8, 8, 128)


def _similarity(query, prototypes):
    return pl.pallas_call(
        _sim_body,
        grid=(B // TC_BLK,),
        in_specs=[
            pl.BlockSpec((TC_BLK, D), lambda i: (i, 0)),
            pl.BlockSpec((P, D), lambda i: (0, 0)),
        ],
        out_specs=pl.BlockSpec((TC_BLK // 8, P // 128, 8, 128),
                               lambda i: (i, 0, 0, 0)),
        out_shape=jax.ShapeDtypeStruct((B // 8, P // 128, 8, 128),
                                       jnp.float32),
    )(query, prototypes)


@functools.partial(
    pl.kernel,
    out_type=jax.ShapeDtypeStruct((B, D), jnp.float32),
    mesh=plsc.VectorSubcoreMesh(core_axis_name="c", subcore_axis_name="s"),
    compiler_params=pltpu.CompilerParams(
        needs_layout_passes=False, use_tc_tiling_on_sc=False),
    scratch_types=[
        pltpu.VMEM((P, D), jnp.float32),
        pltpu.VMEM((_CT, P // 128, 8, 128), jnp.float32),
        pltpu.VMEM((_CT, P // 128, 8, 128), jnp.float32),
        pltpu.VMEM((_CH, D), jnp.float32),
        pltpu.VMEM((_CH, D), jnp.float32),
        pltpu.SemaphoreType.DMA,
        pltpu.SemaphoreType.DMA,
        pltpu.SemaphoreType.DMA,
        pltpu.SemaphoreType.DMA,
    ],
)
def _sc_topk(sim_hbm, protos_hbm, out_hbm, protos_v,
             simv0, simv1, outv0, outv1, si0, si1, so0, so1):
    wid = lax.axis_index("s") * _NC + lax.axis_index("c")
    pltpu.sync_copy(protos_hbm, protos_v)
    lane = lax.broadcasted_iota(jnp.int32, (16,), 0)
    lo8 = lane < 8

    simv = (simv0, simv1)
    outv = (outv0, outv1)
    sin = (si0, si1)
    son = (so0, so1)
    rt_base = wid * (_RPW // 8)

    def make_row_body(sbuf, obuf):
        def row_body(r, _):
            rt = jnp.right_shift(r, 3)
            s = jnp.bitwise_and(r, 7)
            NL = P // 16
            stack = []

            def node_desc(level, idx):
                if level == 5:
                    return True
                return idx % 2 == 0

            for c in range(NL):
                vals = sbuf[rt, c // 8, s, pl.ds((c % 8) * 16, 16)]
                sv, si = plsc.sort_key_val(vals, lane + 16 * c,
                                           descending=node_desc(0, c))
                cur = (0, c, sv, si)
                while stack and stack[-1][0] == cur[0]:
                    lvl, aidx, av, ai = stack.pop()
                    _, _, bv, bi = cur
                    mv = jnp.where(lo8, av, bv)
                    mi = jnp.where(lo8, ai, bi)
                    nl, nidx = lvl + 1, aidx // 2
                    sv, si = plsc.sort_key_val(
                        mv, mi, descending=node_desc(nl, nidx))
                    cur = (nl, nidx, sv, si)
                stack.append(cur)
            v, ix = stack[0][2], stack[0][3]

            e = jnp.where(lo8, jnp.exp(v - v[0]), jnp.float32(0.0))
            w = e / jnp.sum(e)

            accs = [jnp.zeros((16,), jnp.float32) for _ in range(D // 16)]
            for j in range(K):
                pj = jnp.broadcast_to(ix[j], (16,))
                wj = w[j]
                for c in range(D // 16):
                    g = plsc.load_gather(protos_v, [pj, lane + 16 * c])
                    accs[c] = accs[c] + wj * g
            for c in range(D // 16):
                obuf[r, pl.ds(16 * c, 16)] = accs[c]
            return 0
        return row_body

    copies = [pltpu.async_copy(sim_hbm.at[pl.ds(rt_base, _CT)], simv0, si0),
              None]
    out_copies = [None, None]
    for ci in range(_NCH):
        cur = ci % 2
        nxt = 1 - cur
        if ci + 1 < _NCH:
            copies[nxt] = pltpu.async_copy(
                sim_hbm.at[pl.ds(rt_base + (ci + 1) * _CT, _CT)],
                simv[nxt], sin[nxt])
        copies[cur].wait()
        if out_copies[cur] is not None:
            out_copies[cur].wait()
        lax.fori_loop(0, _CH, make_row_body(simv[cur], outv[cur]), 0)
        out_copies[cur] = pltpu.async_copy(
            outv[cur], out_hbm.at[pl.ds(wid * _RPW + ci * _CH, _CH)],
            son[cur])
    for oc in out_copies:
        if oc is not None:
            oc.wait()


@jax.jit
def _run(query, prototypes):
    sim = _similarity(query, prototypes)
    return _sc_topk(sim, prototypes)


def kernel(query, prototypes, k):
    return _run(query, prototypes)
